# baseline (speedup 1.0000x reference)
import dataclasses
import functools

import jax
import jax.numpy as jnp
from jax import lax
from jax.experimental import pallas as pl
from jax.experimental.pallas import tpu as pltpu
from jax.experimental.pallas import tpu_sc as plsc

N1 = 10000
N2 = 1024
D = 128
D_OUT = 41

NC = 2
NS = 16
NW = NC * NS
L = 16
CHUNK = 128
K = 4


def _pad128(n: int) -> int:
  return -(-n // 128) * 128


def _make_sc_segsum(e_pad: int, n_dst: int, c0: int, c1: int):
  assert (c0 + c1) * NS * CHUNK == e_pad
  assert c0 % 4 == 0 and c1 % 4 == 0 and c0 >= 8 and c1 >= 8
  n_pad = _pad128(n_dst)
  crows = n_pad // 128
  zblk = 64
  n_blocks = n_pad // zblk

  mesh = plsc.VectorSubcoreMesh(
      core_axis_name="c", subcore_axis_name="s", num_cores=NC, num_subcores=NS
  )
  cp = pltpu.CompilerParams()
  if "needs_layout_passes" in pltpu.CompilerParams.__dataclass_fields__:
    cp = dataclasses.replace(cp, needs_layout_passes=False)

  @functools.partial(
      pl.kernel,
      compiler_params=cp,
      out_type=(
          jax.ShapeDtypeStruct((NC, n_pad, D), jnp.float32),
          jax.ShapeDtypeStruct((NW, crows, 128), jnp.float32),
      ),
      mesh=mesh,
      scratch_types=[
          pltpu.VMEM((4, 2, CHUNK), jnp.int32),
          [pltpu.VMEM((CHUNK, D), jnp.float32) for _ in range(2)],
          pltpu.VMEM((crows, 128), jnp.float32),
          pltpu.VMEM_SHARED((n_pad, D), jnp.float32),
          [pltpu.SemaphoreType.DMA for _ in range(4)],
          [pltpu.SemaphoreType.DMA for _ in range(2)],
      ],
  )
  def segsum(edge_hbm, table_hbm, zeros_hbm,
             sum_hbm, cnt_hbm, eidx, rows, counts, acc, isem, gsem):
    cid = lax.axis_index("c")
    sid = lax.axis_index("s")
    wid = cid * NS + sid

    @pl.loop(0, crows)
    def _(r):
      @pl.loop(0, 128 // L)
      def _(j):
        counts[r, pl.ds(j * L, L)] = jnp.zeros((L,), jnp.float32)

    @pl.loop(sid, n_blocks, step=NS)
    def _(bi):
      r0 = bi * zblk
      pltpu.sync_copy(zeros_hbm.at[pl.ds(r0, zblk)], acc.at[pl.ds(r0, zblk)])

    plsc.subcore_barrier()

    one = jnp.full((L,), 1.0, jnp.float32)

    def load_idx(c, q):
      pltpu.async_copy(edge_hbm.at[wid, c], eidx.at[q], isem[q])

    def process(q, b, c_next, prefetch):
      pltpu.make_async_copy(edge_hbm.at[wid, 0], eidx.at[q], isem[q]).wait()
      pltpu.sync_copy(table_hbm.at[eidx.at[q, 0]], rows[b])
      for j in range(CHUNK // L):
        dv = eidx[q, 1, pl.ds(j * L, L)]
        row = jax.lax.shift_right_logical(dv, 7)
        lane = jax.lax.bitwise_and(dv, 127)
        plsc.addupdate_scatter(counts, [row, lane], one)
      pltpu.sync_copy(rows[b], acc.at[eidx.at[q, 1]], add=True)
      if prefetch:
        load_idx(c_next, q)

    nck = jnp.where(cid == 0, c0, c1)

    for q in range(4):
      load_idx(q, q)

    @pl.loop(0, nck // 4 - 1)
    def _(t):
      cb = t * 4
      for u in range(4):
        process(u, u % 2, cb + u + 4, True)

    for u in range(4):
      process(u, u % 2, 0, False)

    plsc.subcore_barrier()

    @pl.loop(sid, n_blocks, step=NS)
    def _(bi):
      r0 = bi * zblk
      pltpu.sync_copy(acc.at[pl.ds(r0, zblk)],
                      sum_hbm.at[cid, pl.ds(r0, zblk)])

    pltpu.sync_copy(counts, cnt_hbm.at[wid])

  return segsum


def _dense1_body(s_ref, c_ref, xt_ref, wl_ref, wr_ref, b_ref, o_ref):
  s = s_ref[0] + s_ref[1]
  cnt = jnp.sum(c_ref[...], axis=0)[:, None]
  mean = s / jnp.maximum(cnt, 1.0)
  h = (
      jax.lax.dot(mean, wl_ref[...], precision=lax.Precision.HIGHEST,
                  preferred_element_type=jnp.float32)
      + jax.lax.dot(xt_ref[...], wr_ref[...], precision=lax.Precision.HIGHEST,
                    preferred_element_type=jnp.float32)
      + b_ref[...]
  )
  o_ref[...] = jnp.maximum(h, 0.0)


def _dense2_body(s_ref, c_ref, xt_ref, wl_ref, wr_ref, b_ref, o_ref):
  s = s_ref[0] + s_ref[1]
  cnt = jnp.sum(c_ref[...], axis=0)[:, None]
  mean = s / jnp.maximum(cnt, 1.0)
  o = (
      jax.lax.dot(mean, wl_ref[...], precision=lax.Precision.HIGHEST,
                  preferred_element_type=jnp.float32)
      + jax.lax.dot(xt_ref[...], wr_ref[...], precision=lax.Precision.HIGHEST,
                    preferred_element_type=jnp.float32)
      + b_ref[...]
  )
  z = o - jnp.max(o, axis=-1, keepdims=True)
  o_ref[...] = z - jnp.log(jnp.sum(jnp.exp(z), axis=-1, keepdims=True))


def _dense1(sums, cnts, x_t, wl, wr, b):
  n = x_t.shape[0]
  return pl.pallas_call(
      _dense1_body,
      in_specs=[
          pl.BlockSpec((NC, n, D), lambda: (0, 0, 0)),
          pl.BlockSpec((NW, n), lambda: (0, 0)),
          pl.BlockSpec((n, D), lambda: (0, 0)),
          pl.BlockSpec((D, D), lambda: (0, 0)),
          pl.BlockSpec((D, D), lambda: (0, 0)),
          pl.BlockSpec((1, D), lambda: (0, 0)),
      ],
      out_specs=pl.BlockSpec((n, D), lambda: (0, 0)),
      out_shape=jax.ShapeDtypeStruct((n, D), jnp.float32),
  )(sums, cnts, x_t, wl, wr, b)


def _dense2(sums, cnts, x_t, wl, wr, b):
  return pl.pallas_call(
      _dense2_body,
      in_specs=[
          pl.BlockSpec((NC, N2, D), lambda: (0, 0, 0)),
          pl.BlockSpec((NW, N2), lambda: (0, 0)),
          pl.BlockSpec((N2, D), lambda: (0, 0)),
          pl.BlockSpec((D, D_OUT), lambda: (0, 0)),
          pl.BlockSpec((D, D_OUT), lambda: (0, 0)),
          pl.BlockSpec((1, D_OUT), lambda: (0, 0)),
      ],
      out_specs=pl.BlockSpec((N2, D_OUT), lambda: (0, 0)),
      out_shape=jax.ShapeDtypeStruct((N2, D_OUT), jnp.float32),
  )(sums, cnts, x_t, wl, wr, b)


E1, E2 = 320000, 32768
_EBLK = NW * CHUNK * K
_E1_PAD = _EBLK * (-(-E1 // _EBLK))
_E2_PAD = _EBLK * (-(-E2 // _EBLK))
_C0_1, _C1_1 = 80, 80
_C0_2, _C1_2 = 8, 8
_segsum1 = _make_sc_segsum(e_pad=_E1_PAD, n_dst=N1, c0=_C0_1, c1=_C1_1)
_segsum2 = _make_sc_segsum(e_pad=_E2_PAD, n_dst=N2, c0=_C0_2, c1=_C1_2)


def _pad_edges(src, dst, e_pad, trash_lo, trash_n, c0, c1):
  e = src.shape[0]
  if e_pad != e:
    trash = trash_lo + jnp.arange(e_pad - e, dtype=dst.dtype) % trash_n
    src = jnp.concatenate([src, jnp.zeros((e_pad - e,), src.dtype)])
    dst = jnp.concatenate([dst, trash])
  ch = jnp.stack([src.reshape(-1, CHUNK), dst.reshape(-1, CHUNK)], axis=1)
  a0 = ch[: NS * c0].reshape(NS, c0, 2, CHUNK)
  a1 = ch[NS * c0:].reshape(NS, c1, 2, CHUNK)
  if c1 < c0:
    a1 = jnp.pad(a1, ((0, 0), (0, c0 - c1), (0, 0), (0, 0)))
  return jnp.concatenate([a0, a1], axis=0)


def kernel(x, edge_index1, edge_index2, W1_l, W1_r, b1, W2_l, W2_r, b2,
           n_target1, n_target2):
  n1_pad = _pad128(N1)
  x_t = lax.dynamic_slice_in_dim(x, n_target1 - N1, N1, axis=0)
  x_t = jnp.concatenate([x_t, jnp.zeros((n1_pad - N1, D), jnp.float32)])
  zeros1 = jnp.zeros((n1_pad, D), jnp.float32)
  edges1 = _pad_edges(edge_index1[0], edge_index1[1], _E1_PAD,
                      N1, n1_pad - N1, _C0_1, _C1_1)
  sum1, cntw1 = _segsum1(edges1, x, zeros1)
  cnt1 = cntw1.reshape(NW, n1_pad)
  h = _dense1(sum1, cnt1, x_t, W1_l, W1_r, b1.reshape(1, D))

  h_t = lax.dynamic_slice_in_dim(h, n_target2 - N2, N2, axis=0)
  zeros2 = jnp.zeros((N2, D), jnp.float32)
  edges2 = _pad_edges(edge_index2[0], edge_index2[1], _E2_PAD, N2 - 1, 1,
                      _C0_2, _C1_2)
  sum2, cntw2 = _segsum2(edges2, h, zeros2)
  cnt2 = cntw2.reshape(NW, N2)
  return _dense2(sum2, cnt2, h_t, W2_l, W2_r, b2.reshape(1, D_OUT))


# speedup vs baseline: 1.3825x; 1.3825x over previous
import dataclasses
import functools

import jax
import jax.numpy as jnp
from jax import lax
from jax.experimental import pallas as pl
from jax.experimental.pallas import tpu as pltpu
from jax.experimental.pallas import tpu_sc as plsc

N1 = 10000
N2 = 1024
D = 128
D_OUT = 41

NC = 2
NS = 16
NW = NC * NS
L = 16


def _pad128(n: int) -> int:
  return -(-n // 128) * 128


def _make_sc_segsum(e_total: int, n_dst: int, chunk: int):
  edges_per_tile = e_total // NW
  n_chunks = edges_per_tile // chunk
  assert edges_per_tile % chunk == 0 and chunk % 8 == 0 and chunk <= 128
  n_pad = _pad128(n_dst)
  crows = n_pad // 128
  zblk = 64
  n_blocks = n_pad // zblk

  mesh = plsc.VectorSubcoreMesh(
      core_axis_name="c", subcore_axis_name="s", num_cores=NC, num_subcores=NS
  )
  cp = pltpu.CompilerParams()
  if "needs_layout_passes" in pltpu.CompilerParams.__dataclass_fields__:
    cp = dataclasses.replace(cp, needs_layout_passes=False)

  @functools.partial(
      pl.kernel,
      compiler_params=cp,
      out_type=(
          jax.ShapeDtypeStruct((NC, n_pad, D), jnp.float32),
          jax.ShapeDtypeStruct((NW, crows, 128), jnp.float32),
      ),
      mesh=mesh,
      scratch_types=[
          pltpu.VMEM((chunk,), jnp.int32),
          pltpu.VMEM((chunk,), jnp.int32),
          pltpu.VMEM((chunk, D), jnp.float32),
          pltpu.VMEM((crows, 128), jnp.float32),
          pltpu.VMEM_SHARED((n_pad, D), jnp.float32),
      ],
  )
  def segsum(src_hbm, dst_hbm, table_hbm, zeros_hbm,
             sum_hbm, cnt_hbm, sidx, didx, rows, counts, acc):
    cid = lax.axis_index("c")
    sid = lax.axis_index("s")
    wid = cid * NS + sid

    @pl.loop(0, crows)
    def _(r):
      @pl.loop(0, 128 // L)
      def _(j):
        counts[r, pl.ds(j * L, L)] = jnp.zeros((L,), jnp.float32)

    @pl.loop(sid, n_blocks, step=NS)
    def _(bi):
      r0 = bi * zblk
      pltpu.sync_copy(zeros_hbm.at[pl.ds(r0, zblk)], acc.at[pl.ds(r0, zblk)])

    plsc.subcore_barrier()

    base = cid * (e_total // NC) + sid * edges_per_tile
    one = jnp.full((L,), 1.0, jnp.float32)

    @pl.loop(0, n_chunks)
    def _(i):
      eb = base + i * chunk
      pltpu.sync_copy(src_hbm.at[pl.ds(eb, chunk)], sidx)
      pltpu.sync_copy(dst_hbm.at[pl.ds(eb, chunk)], didx)
      pltpu.sync_copy(table_hbm.at[sidx], rows)
      for j in range(chunk // L):
        dv = didx[pl.ds(j * L, L)]
        row = jax.lax.shift_right_logical(dv, 7)
        lane = jax.lax.bitwise_and(dv, 127)
        plsc.addupdate_scatter(counts, [row, lane], one)
      pltpu.sync_copy(rows, acc.at[didx], add=True)

    plsc.subcore_barrier()

    @pl.loop(sid, n_blocks, step=NS)
    def _(bi):
      r0 = bi * zblk
      pltpu.sync_copy(acc.at[pl.ds(r0, zblk)],
                      sum_hbm.at[cid, pl.ds(r0, zblk)])

    pltpu.sync_copy(counts, cnt_hbm.at[wid])

  return segsum


def _dense1_body(s_ref, c_ref, xt_ref, wl_ref, wr_ref, b_ref, o_ref):
  s = s_ref[0] + s_ref[1]
  cnt = jnp.sum(c_ref[...], axis=0)[:, None]
  mean = s / jnp.maximum(cnt, 1.0)
  h = (
      jax.lax.dot(mean, wl_ref[...], precision=lax.Precision.HIGHEST,
                  preferred_element_type=jnp.float32)
      + jax.lax.dot(xt_ref[...], wr_ref[...], precision=lax.Precision.HIGHEST,
                    preferred_element_type=jnp.float32)
      + b_ref[...]
  )
  o_ref[...] = jnp.maximum(h, 0.0)


def _dense2_body(s_ref, c_ref, xt_ref, wl_ref, wr_ref, b_ref, o_ref):
  s = s_ref[0] + s_ref[1]
  cnt = jnp.sum(c_ref[...], axis=0)[:, None]
  mean = s / jnp.maximum(cnt, 1.0)
  o = (
      jax.lax.dot(mean, wl_ref[...], precision=lax.Precision.HIGHEST,
                  preferred_element_type=jnp.float32)
      + jax.lax.dot(xt_ref[...], wr_ref[...], precision=lax.Precision.HIGHEST,
                    preferred_element_type=jnp.float32)
      + b_ref[...]
  )
  z = o - jnp.max(o, axis=-1, keepdims=True)
  o_ref[...] = z - jnp.log(jnp.sum(jnp.exp(z), axis=-1, keepdims=True))


def _dense1(sums, cnts, x_t, wl, wr, b):
  n = x_t.shape[0]
  return pl.pallas_call(
      _dense1_body,
      in_specs=[
          pl.BlockSpec((NC, n, D), lambda: (0, 0, 0)),
          pl.BlockSpec((NW, n), lambda: (0, 0)),
          pl.BlockSpec((n, D), lambda: (0, 0)),
          pl.BlockSpec((D, D), lambda: (0, 0)),
          pl.BlockSpec((D, D), lambda: (0, 0)),
          pl.BlockSpec((1, D), lambda: (0, 0)),
      ],
      out_specs=pl.BlockSpec((n, D), lambda: (0, 0)),
      out_shape=jax.ShapeDtypeStruct((n, D), jnp.float32),
  )(sums, cnts, x_t, wl, wr, b)


def _dense2(sums, cnts, x_t, wl, wr, b):
  return pl.pallas_call(
      _dense2_body,
      in_specs=[
          pl.BlockSpec((NC, N2, D), lambda: (0, 0, 0)),
          pl.BlockSpec((NW, N2), lambda: (0, 0)),
          pl.BlockSpec((N2, D), lambda: (0, 0)),
          pl.BlockSpec((D, D_OUT), lambda: (0, 0)),
          pl.BlockSpec((D, D_OUT), lambda: (0, 0)),
          pl.BlockSpec((1, D_OUT), lambda: (0, 0)),
      ],
      out_specs=pl.BlockSpec((N2, D_OUT), lambda: (0, 0)),
      out_shape=jax.ShapeDtypeStruct((N2, D_OUT), jnp.float32),
  )(sums, cnts, x_t, wl, wr, b)


_segsum1 = _make_sc_segsum(e_total=320000, n_dst=N1, chunk=80)
_segsum2 = _make_sc_segsum(e_total=32768, n_dst=N2, chunk=128)


def kernel(x, edge_index1, edge_index2, W1_l, W1_r, b1, W2_l, W2_r, b2,
           n_target1, n_target2):
  n1_pad = _pad128(N1)
  x_t = lax.dynamic_slice_in_dim(x, n_target1 - N1, N1, axis=0)
  x_t = jnp.concatenate([x_t, jnp.zeros((n1_pad - N1, D), jnp.float32)])
  zeros1 = jnp.zeros((n1_pad, D), jnp.float32)
  sum1, cntw1 = _segsum1(edge_index1[0], edge_index1[1], x, zeros1)
  cnt1 = cntw1.reshape(NW, n1_pad)
  h = _dense1(sum1, cnt1, x_t, W1_l, W1_r, b1.reshape(1, D))

  h_t = lax.dynamic_slice_in_dim(h, n_target2 - N2, N2, axis=0)
  zeros2 = jnp.zeros((N2, D), jnp.float32)
  sum2, cntw2 = _segsum2(edge_index2[0], edge_index2[1], h, zeros2)
  cnt2 = cntw2.reshape(NW, N2)
  return _dense2(sum2, cnt2, h_t, W2_l, W2_r, b2.reshape(1, D_OUT))
